# baseline (speedup 1.0000x reference)
import jax
import jax.numpy as jnp
from jax import lax
from jax.experimental import pallas as pl
from jax.experimental.pallas import tpu as pltpu

_N = 8192
_D = 64
_H = 16
_K = 20
_B = 256
_C = 128
_NC = _N // _C
_P = 4

_BIG = 3.0e38
_LOG2E = 1.4426950408889634


def _full_threshold(g):
    cur = g
    t = None
    for _ in range(_K):
        t = jnp.min(cur, axis=1, keepdims=True)
        cur = jnp.where(cur <= t, _BIG, cur)
    return t


def _pool_threshold(pool):
    cur = list(pool)
    t = None
    for _ in range(_K):
        m = cur[0]
        for c in cur[1:]:
            m = jnp.minimum(m, c)
        t = jnp.min(m, axis=1, keepdims=True)
        cur = [jnp.where(c <= t, _BIG, c) for c in cur]
    return t


def _body(x_ref, wmat_ref, out_ref, xa_ref, fa_ref, f_ref):
    i = pl.program_id(0)

    @pl.when(i == 0)
    def _init():
        f = jnp.dot(x_ref[...], wmat_ref[...].T,
                    preferred_element_type=jnp.float32)
        f_ref[...] = f
        ft = f.T
        sq = jnp.sum(ft * ft, axis=0, keepdims=True)
        fa_ref[...] = jnp.concatenate([ft, sq], axis=0)
        xa_ref[...] = jnp.concatenate(
            [x_ref[...], jnp.ones((_N, 1), jnp.float32)], axis=1)

    fb = f_ref[pl.ds(i * _B, _B), :]
    dot = jnp.dot(fb, fa_ref[:_H, :], preferred_element_type=jnp.float32)
    sq_row = fa_ref[_H:_H + 1, :]
    sq_col = fa_ref[_H, pl.ds(i * _B, _B)][:, None]
    h = sq_row - 2.0 * dot

    rs = h.reshape(_B, _NC, _C)
    pool = []
    m = jnp.min(rs, axis=1)
    pool.append(m)
    for _ in range(_P - 1):
        m = jnp.min(jnp.where(rs > m[:, None, :], rs, _BIG), axis=1)
        pool.append(m)

    t = _pool_threshold(pool)

    cnt = jnp.sum(jnp.where(h <= t, 1.0, 0.0), axis=1, keepdims=True)
    ok = jnp.all(cnt == float(_K))
    t = lax.cond(ok, lambda: t, lambda: _full_threshold(h))

    w = jnp.where(h <= t,
                  jnp.exp2(jnp.maximum(h + sq_col, 0.0) * (-_LOG2E)),
                  0.0)
    acc = jnp.dot(w, xa_ref[...], preferred_element_type=jnp.float32)
    out_ref[...] = acc[:, :_D] / acc[:, _D:_D + 1]


def kernel(x, y, W):
    del y
    out = pl.pallas_call(
        _body,
        grid=(_N // _B,),
        in_specs=[
            pl.BlockSpec((_N, _D), lambda i: (0, 0)),
            pl.BlockSpec((_H, _D), lambda i: (0, 0)),
        ],
        out_specs=pl.BlockSpec((_B, _D), lambda i: (i, 0)),
        out_shape=jax.ShapeDtypeStruct((_N, _D), jnp.float32),
        scratch_shapes=[
            pltpu.VMEM((_N, _D + 1), jnp.float32),
            pltpu.VMEM((_H + 1, _N), jnp.float32),
            pltpu.VMEM((_N, _H), jnp.float32),
        ],
        compiler_params=pltpu.CompilerParams(
            dimension_semantics=("arbitrary",),
        ),
    )(x, W)
    return out


# speedup vs baseline: 1.7354x; 1.7354x over previous
import jax
import jax.numpy as jnp
from jax import lax
from jax.experimental import pallas as pl
from jax.experimental.pallas import tpu as pltpu

_N = 8192
_D = 64
_H = 16
_K = 20
_B = 256
_C = 128
_NC = _N // _C
_P = 6

_BIG = 3.0e38
_LOG2E = 1.4426950408889634


def _full_threshold(g):
    cur = g
    t = None
    for _ in range(_K):
        t = jnp.min(cur, axis=1, keepdims=True)
        cur = jnp.where(cur <= t, _BIG, cur)
    return t


def _body(x_ref, wmat_ref, out_ref, xa_ref, fa_ref, f_ref):
    i = pl.program_id(0)

    @pl.when(i == 0)
    def _init():
        f = jnp.dot(x_ref[...], wmat_ref[...].T,
                    preferred_element_type=jnp.float32)
        f_ref[...] = f
        ft = f.T
        sq = jnp.sum(ft * ft, axis=0, keepdims=True)
        fa_ref[...] = jnp.concatenate([ft, sq], axis=0)
        xa_ref[...] = jnp.concatenate(
            [x_ref[...], jnp.ones((_N, 1), jnp.float32)], axis=1)

    fb = f_ref[pl.ds(i * _B, _B), :]
    dot = jnp.dot(fb, fa_ref[:_H, :], preferred_element_type=jnp.float32)
    sq_row = fa_ref[_H:_H + 1, :]
    sq_col = fa_ref[_H, pl.ds(i * _B, _B)][:, None]
    h = sq_row - 2.0 * dot

    rs = h.reshape(_B, _NC, _C)
    pool = []
    m = jnp.min(rs, axis=1)
    pool.append(m)
    for _ in range(_P - 1):
        m = jnp.min(jnp.where(rs > m[:, None, :], rs, _BIG), axis=1)
        pool.append(m)
    pv = jnp.concatenate(pool, axis=1)

    t = _full_threshold(pv)

    ok = jnp.all(pool[-1] > t)
    t = lax.cond(ok, lambda: t, lambda: _full_threshold(h))

    w = jnp.where(h <= t,
                  jnp.exp2(jnp.maximum(h + sq_col, 0.0) * (-_LOG2E)),
                  0.0)
    acc = jnp.dot(w, xa_ref[...], preferred_element_type=jnp.float32)
    out_ref[...] = acc[:, :_D] / acc[:, _D:_D + 1]


def kernel(x, y, W):
    del y
    out = pl.pallas_call(
        _body,
        grid=(_N // _B,),
        in_specs=[
            pl.BlockSpec((_N, _D), lambda i: (0, 0)),
            pl.BlockSpec((_H, _D), lambda i: (0, 0)),
        ],
        out_specs=pl.BlockSpec((_B, _D), lambda i: (i, 0)),
        out_shape=jax.ShapeDtypeStruct((_N, _D), jnp.float32),
        scratch_shapes=[
            pltpu.VMEM((_N, _D + 1), jnp.float32),
            pltpu.VMEM((_H + 1, _N), jnp.float32),
            pltpu.VMEM((_N, _H), jnp.float32),
        ],
        compiler_params=pltpu.CompilerParams(
            dimension_semantics=("arbitrary",),
        ),
    )(x, W)
    return out


# speedup vs baseline: 1.8010x; 1.0378x over previous
import jax
import jax.numpy as jnp
from jax import lax
from jax.experimental import pallas as pl
from jax.experimental.pallas import tpu as pltpu

_N = 8192
_D = 64
_H = 16
_K = 20
_B = 256
_C = 128
_NC = _N // _C
_P = 5

_BIG = 3.0e38
_LOG2E = 1.4426950408889634


def _full_threshold(g):
    cur = g
    t = None
    for _ in range(_K):
        t = jnp.min(cur, axis=1, keepdims=True)
        cur = jnp.where(cur <= t, _BIG, cur)
    return t


def _body(x_ref, wmat_ref, out_ref, xa_ref, fa_ref, f_ref):
    i = pl.program_id(0)

    @pl.when(i == 0)
    def _init():
        f = jnp.dot(x_ref[...], wmat_ref[...].T,
                    preferred_element_type=jnp.float32)
        f_ref[...] = f
        ft = f.T
        sq = jnp.sum(ft * ft, axis=0, keepdims=True)
        fa_ref[...] = jnp.concatenate([ft, sq], axis=0)
        xa_ref[...] = jnp.concatenate(
            [x_ref[...], jnp.ones((_N, 1), jnp.float32)], axis=1)

    fb = f_ref[pl.ds(i * _B, _B), :]
    dot = jnp.dot(fb, fa_ref[:_H, :], preferred_element_type=jnp.float32)
    sq_row = fa_ref[_H:_H + 1, :]
    sq_col = fa_ref[_H, pl.ds(i * _B, _B)][:, None]
    h = sq_row - 2.0 * dot

    rs = h.reshape(_B, _NC, _C)
    pool = []
    m = jnp.min(rs, axis=1)
    pool.append(m)
    for _ in range(_P - 1):
        m = jnp.min(jnp.where(rs > m[:, None, :], rs, _BIG), axis=1)
        pool.append(m)
    pv = jnp.concatenate(pool, axis=1)

    t = _full_threshold(pv)

    ok = jnp.all(pool[-1] > t)
    t = lax.cond(ok, lambda: t, lambda: _full_threshold(h))

    w = jnp.where(h <= t,
                  jnp.exp2(jnp.maximum(h + sq_col, 0.0) * (-_LOG2E)),
                  0.0)
    acc = jnp.dot(w, xa_ref[...], preferred_element_type=jnp.float32)
    out_ref[...] = acc[:, :_D] / acc[:, _D:_D + 1]


def kernel(x, y, W):
    del y
    out = pl.pallas_call(
        _body,
        grid=(_N // _B,),
        in_specs=[
            pl.BlockSpec((_N, _D), lambda i: (0, 0)),
            pl.BlockSpec((_H, _D), lambda i: (0, 0)),
        ],
        out_specs=pl.BlockSpec((_B, _D), lambda i: (i, 0)),
        out_shape=jax.ShapeDtypeStruct((_N, _D), jnp.float32),
        scratch_shapes=[
            pltpu.VMEM((_N, _D + 1), jnp.float32),
            pltpu.VMEM((_H + 1, _N), jnp.float32),
            pltpu.VMEM((_N, _H), jnp.float32),
        ],
        compiler_params=pltpu.CompilerParams(
            dimension_semantics=("arbitrary",),
        ),
    )(x, W)
    return out
